# baseline (speedup 1.0000x reference)
import functools

import jax
import jax.numpy as jnp
from jax.experimental import pallas as pl
from jax.experimental.pallas import tpu as pltpu
from jax.experimental.pallas import tpu_sc as plsc

_NUM_LAYERS = 2
_S_LEN = 200
_B = 1024
_D = 64
_W = 128
_BW = _B // _W


def _sc_gather(table, idx):
    mesh = plsc.VectorSubcoreMesh(
        core_axis_name="core", subcore_axis_name="subcore"
    )

    @functools.partial(
        pl.kernel,
        out_type=jax.ShapeDtypeStruct((_S_LEN, _B, _D), jnp.float32),
        mesh=mesh,
        compiler_params=pltpu.CompilerParams(use_tc_tiling_on_sc=False),
    )
    def k(table_hbm, idx_hbm, out_hbm):
        def body(i_vmem, o_vmem):
            pltpu.sync_copy(table_hbm.at[i_vmem.at[0]], o_vmem.at[0])

        pltpu.emit_pipeline(
            body,
            grid=(_S_LEN * _BW,),
            in_specs=[pl.BlockSpec((1, _W), lambda i: (i // _BW, i % _BW))],
            out_specs=[
                pl.BlockSpec((1, _W, _D), lambda i: (i // _BW, i % _BW, 0))
            ],
            core_axis_name=("core", "subcore"),
            dimension_semantics=(pltpu.PARALLEL,),
        )(idx_hbm, out_hbm)

    return k(table, idx)


_QBLK = 128


def _tc_mean(emb):
    def body(x_ref, o_ref):
        s = jnp.sum(x_ref[...], axis=0) * (1.0 / _S_LEN)
        o_ref[...] = jnp.broadcast_to(s[None], (_NUM_LAYERS, _QBLK, _D))

    return pl.pallas_call(
        body,
        grid=(_B // _QBLK,),
        in_specs=[pl.BlockSpec((_S_LEN, _QBLK, _D), lambda i: (0, i, 0))],
        out_specs=pl.BlockSpec((_NUM_LAYERS, _QBLK, _D), lambda i: (0, i, 0)),
        out_shape=jax.ShapeDtypeStruct((_NUM_LAYERS, _B, _D), jnp.float32),
    )(emb)


def kernel(src, lengths, table):
    del lengths
    idx = src[..., 0].astype(jnp.int32)
    memory_bank = _sc_gather(table, idx)
    enc_final = _tc_mean(memory_bank)
    return (enc_final, enc_final, memory_bank)


# speedup vs baseline: 1.0236x; 1.0236x over previous
import functools

import jax
import jax.numpy as jnp
from jax.experimental import pallas as pl
from jax.experimental.pallas import tpu as pltpu
from jax.experimental.pallas import tpu_sc as plsc

_NUM_LAYERS = 2
_S_LEN = 200
_B = 1024
_D = 64
_W = 128
_BW = _B // _W


def _sc_gather(table, idx):
    mesh = plsc.VectorSubcoreMesh(
        core_axis_name="core", subcore_axis_name="subcore"
    )

    @functools.partial(
        pl.kernel,
        out_type=jax.ShapeDtypeStruct((_S_LEN, _B, _D), jnp.float32),
        mesh=mesh,
        compiler_params=pltpu.CompilerParams(use_tc_tiling_on_sc=False),
    )
    def k(table_hbm, idx_hbm, out_hbm):
        def body(i_vmem, o_vmem):
            pltpu.sync_copy(table_hbm.at[i_vmem.at[0]], o_vmem.at[0])

        pltpu.emit_pipeline(
            body,
            grid=(_S_LEN * _BW,),
            in_specs=[pl.BlockSpec((1, _W), lambda i: (i // _BW, i % _BW))],
            out_specs=[
                pl.BlockSpec((1, _W, _D), lambda i: (i // _BW, i % _BW, 0))
            ],
            core_axis_name=("core", "subcore"),
            dimension_semantics=(pltpu.PARALLEL,),
        )(idx_hbm, out_hbm)

    return k(table, idx)


_QBLK = 128


def _tc_mean_copy(emb, table):
    def body(x_ref, t_ref, mb_ref, e1_ref, e2_ref):
        del t_ref
        x = x_ref[...]
        xt = jnp.swapaxes(x, 1, 2)
        mb_ref[...] = xt
        s = jnp.sum(xt, axis=0) * (1.0 / _S_LEN)
        e = jnp.broadcast_to(s[None], (_NUM_LAYERS, _D, _QBLK))
        e1_ref[...] = e
        e2_ref[...] = e

    return pl.pallas_call(
        body,
        grid=(_B // _QBLK,),
        in_specs=[
            pl.BlockSpec((_S_LEN, _QBLK, _D), lambda i: (0, i, 0)),
            pl.BlockSpec((8, _D), lambda i: (0, 0)),
        ],
        out_specs=[
            pl.BlockSpec((_S_LEN, _D, _QBLK), lambda i: (0, 0, i)),
            pl.BlockSpec((_NUM_LAYERS, _D, _QBLK), lambda i: (0, 0, i)),
            pl.BlockSpec((_NUM_LAYERS, _D, _QBLK), lambda i: (0, 0, i)),
        ],
        out_shape=[
            jax.ShapeDtypeStruct((_S_LEN, _D, _B), jnp.float32),
            jax.ShapeDtypeStruct((_NUM_LAYERS, _D, _B), jnp.float32),
            jax.ShapeDtypeStruct((_NUM_LAYERS, _D, _B), jnp.float32),
        ],
    )(emb, table)


def kernel(src, lengths, table):
    del lengths
    idx = src[..., 0].astype(jnp.int32)
    gathered = _sc_gather(table, idx)
    mb_t, ef1_t, ef2_t = _tc_mean_copy(gathered, table)
    memory_bank = jnp.swapaxes(mb_t, 1, 2)
    ef1 = jnp.swapaxes(ef1_t, 1, 2)
    ef2 = jnp.swapaxes(ef2_t, 1, 2)
    return (ef1, ef2, memory_bank)
